# baseline (speedup 1.0000x reference)
import functools
import math

import jax
import jax.numpy as jnp
from jax import lax
from jax.experimental import pallas as pl
from jax.experimental.pallas import tpu as pltpu
from jax.experimental.pallas import tpu_sc as plsc

N = 10000
E = 160000
NGRAPH = 16
NB = 10
MAX_RADIUS = 3.5
D_ATTR = 16

NP = 10240
CHP = 128
NCHUNK_P = E // CHP
NC = 2
NS = 16
ROWS_PER_TILE = NP // NS

DIMS = [128, 144, 144, 144, 64]
NLAYERS = 4
DH = {144: 80, 64: 32}
DREAL = {144: 72, 64: 32}
DW = 128

CHS = 80
NCHUNK_S = E // CHS
NKS = NCHUNK_S // NS


def _sc_mesh():
    return plsc.VectorSubcoreMesh(core_axis_name="c", subcore_axis_name="s")


_SC_PARAMS = pltpu.CompilerParams(use_tc_tiling_on_sc=False)
_SC_PARAMS_NL = pltpu.CompilerParams(use_tc_tiling_on_sc=False,
                                     needs_layout_passes=False)
_BF = jnp.bfloat16


def _split_pad_cols(w, do, dh):
    dr = DREAL[do]
    a = w[..., :dr]
    b = w[..., dr:do]
    pad = [(0, 0)] * (w.ndim - 1)
    a = jnp.pad(a, pad + [(0, dh - a.shape[-1])])
    b = jnp.pad(b, pad + [(0, dh - b.shape[-1])])
    return a, b


def _swizzle(w, dhc):
    perm = []
    for g in range(dhc // 32):
        for k in range(16):
            perm.append(32 * g + k)
            perm.append(32 * g + 16 + k)
    return w[..., jnp.array(perm, dtype=jnp.int32)]


NKP = NCHUNK_P // 32
REMP = NCHUNK_P - NKP * 32


def _make_pos_gather():
    @functools.partial(
        pl.kernel,
        out_type=jax.ShapeDtypeStruct((E, 16), jnp.float32),
        mesh=_sc_mesh(),
        scratch_types=[
            pltpu.VMEM((NKP + 1, CHP), jnp.int32),
            pltpu.VMEM((NKP + 1, CHP), jnp.int32),
            pltpu.VMEM((CHP, 16), jnp.float32),
            pltpu.VMEM((CHP, 16), jnp.float32),
            pltpu.VMEM((CHP, 16), jnp.float32),
            pltpu.VMEM((CHP, 16), jnp.float32),
            pltpu.VMEM((CHP, 16), jnp.float32),
            pltpu.VMEM((CHP, 16), jnp.float32),
            pltpu.SemaphoreType.DMA,
            pltpu.SemaphoreType.DMA,
            pltpu.SemaphoreType.DMA,
            pltpu.SemaphoreType.DMA,
            pltpu.SemaphoreType.DMA,
            pltpu.SemaphoreType.DMA,
        ],
        compiler_params=_SC_PARAMS,
    )
    def k(pos16, srcs2, dsts2, ev_out, src_all, dst_all,
          a0, a1, b0, b1, e0, e1, ga0, ga1, gb0, gb1, w0, w1):
        c = lax.axis_index("c")
        s = lax.axis_index("s")
        wid = s * NC + c
        abuf = (a0, a1)
        bbuf = (b0, b1)
        ebuf = (e0, e1)
        gsa = (ga0, ga1)
        gsb = (gb0, gb1)
        wsem = (w0, w1)
        cbase = wid * NKP + jnp.minimum(wid, REMP)
        nk = NKP + jnp.where(wid < REMP, 1, 0)
        pltpu.sync_copy(srcs2.at[pl.ds(cbase, NKP)],
                        src_all.at[pl.ds(0, NKP)])
        pltpu.sync_copy(dsts2.at[pl.ds(cbase, NKP)],
                        dst_all.at[pl.ds(0, NKP)])

        @pl.when(nk > NKP)
        def _():
            pltpu.sync_copy(srcs2.at[pl.ds(cbase + NKP, 1)],
                            src_all.at[pl.ds(NKP, 1)])
            pltpu.sync_copy(dsts2.at[pl.ds(cbase + NKP, 1)],
                            dst_all.at[pl.ds(NKP, 1)])

        def issue(kk, p):
            pltpu.async_copy(pos16.at[src_all.at[kk]], abuf[p], gsa[p])
            pltpu.async_copy(pos16.at[dst_all.at[kk]], bbuf[p], gsb[p])

        issue(0, 0)
        issue(1, 1)
        dummy = pos16.at[pl.ds(0, CHP)]

        def pair(q, _):
            for p in range(2):
                kk = q * 2 + p

                @pl.when(kk < nk)
                def _(kk=kk, p=p):
                    pltpu.make_async_copy(dummy, abuf[p], gsa[p]).wait()
                    pltpu.make_async_copy(dummy, bbuf[p], gsb[p]).wait()

                    @pl.when(kk >= 2)
                    def _():
                        pltpu.make_async_copy(dummy, ebuf[p], wsem[p]).wait()

                    @plsc.parallel_loop(0, CHP, 1, unroll=4)
                    def _(i):
                        sl = pl.ds(0, 16)
                        ebuf[p][i, sl] = abuf[p][i, sl] - bbuf[p][i, sl]

                    pltpu.async_copy(
                        ebuf[p], ev_out.at[pl.ds((cbase + kk) * CHP, CHP)],
                        wsem[p])

                    @pl.when(kk + 2 < nk)
                    def _():
                        issue(kk + 2, p)
            return 0

        lax.fori_loop(0, (nk + 1) // 2, pair, 0)
        for p in range(2):
            pltpu.make_async_copy(dummy, ebuf[p], wsem[p]).wait()

    return k


def _make_sc_scatter(dh, dwc):
    @functools.partial(
        pl.kernel,
        out_type=jax.ShapeDtypeStruct((NC, NP, dh), jnp.float32),
        mesh=_sc_mesh(),
        scratch_types=[
            pltpu.VMEM((NKS, CHS), jnp.int32),
            pltpu.VMEM((NKS, CHS), jnp.int32),
            pltpu.VMEM((CHS, DW), jnp.float32),
            pltpu.VMEM((CHS, DW), jnp.float32),
            pltpu.VMEM((CHS, dwc), jnp.float32),
            pltpu.VMEM((CHS, dwc), jnp.float32),
            pltpu.VMEM((CHS, dh), jnp.float32),
            pltpu.VMEM((CHS, dh), jnp.float32),
            pltpu.VMEM((16, dh), jnp.float32),
            pltpu.VMEM_SHARED((NP, dh), jnp.float32),
            pltpu.SemaphoreType.DMA,
            pltpu.SemaphoreType.DMA,
            pltpu.SemaphoreType.DMA,
            pltpu.SemaphoreType.DMA,
            pltpu.SemaphoreType.DMA,
            pltpu.SemaphoreType.DMA,
        ],
        compiler_params=_SC_PARAMS,
    )
    def k(hf2, coef2, srcs2, dsts2, out, src_all, dst_all,
          r0, r1, cl0, cl1, p0, p1, zbuf, aggsh,
          g0, g1, q0, q1, s0, s1):
        c = lax.axis_index("c")
        s = lax.axis_index("s")
        rows = (r0, r1)
        cld = (cl0, cl1)
        prod = (p0, p1)
        gsem = (g0, g1)
        csem = (q0, q1)
        ssem = (s0, s1)
        cbase = s * NKS

        pltpu.sync_copy(srcs2.at[pl.ds(cbase, NKS)], src_all)
        pltpu.sync_copy(dsts2.at[pl.ds(cbase, NKS)], dst_all)

        zero16 = jnp.zeros((16,), jnp.float32)

        def zb(i, _):
            for j in range(dh // 16):
                zbuf[i, pl.ds(j * 16, 16)] = zero16
            return 0

        lax.fori_loop(0, 16, zb, 0)
        for j in range(ROWS_PER_TILE // 16):
            pltpu.sync_copy(zbuf, aggsh.at[pl.ds(s * ROWS_PER_TILE + j * 16, 16)])
        plsc.subcore_barrier()

        def issue(kk, b):
            pltpu.async_copy(hf2.at[c].at[src_all.at[kk]], rows[b], gsem[b])
            pltpu.async_copy(coef2.at[c, pl.ds((cbase + kk) * CHS, CHS)],
                             cld[b], csem[b])

        issue(0, 0)
        issue(1, 1)
        dummy = hf2.at[c, pl.ds(0, CHS)]
        dummyc = coef2.at[c, pl.ds(0, CHS)]
        dummyn = out.at[c, pl.ds(0, CHS)]

        def pair(q, _):
            for b in range(2):
                kk = q * 2 + b

                @pl.when(kk < NKS)
                def _(kk=kk, b=b):
                    pltpu.make_async_copy(dummy, rows[b], gsem[b]).wait()
                    pltpu.make_async_copy(dummyc, cld[b], csem[b]).wait()

                    @pl.when(kk >= 2)
                    def _():
                        pltpu.make_async_copy(dummyn, prod[b], ssem[b]).wait()

                    @plsc.parallel_loop(0, CHS, 1, unroll=4)
                    def _(i):
                        for j in range(dh // 16):
                            sl = pl.ds(j * 16, 16)
                            prod[b][i, sl] = rows[b][i, sl] * cld[b][i, sl]

                    pltpu.async_copy(prod[b], aggsh.at[dst_all.at[kk]],
                                     ssem[b], add=True)

                    @pl.when(kk + 2 < NKS)
                    def _():
                        issue(kk + 2, b)
            return 0

        lax.fori_loop(0, (NKS + 1) // 2, pair, 0)
        for b in range(2):
            pltpu.make_async_copy(dummyn, prod[b], ssem[b]).wait()
        plsc.subcore_barrier()
        for j in range(ROWS_PER_TILE // CHS):
            st = s * ROWS_PER_TILE + j * CHS
            pltpu.sync_copy(aggsh.at[pl.ds(st, CHS)], p0)
            pltpu.sync_copy(p0, out.at[c, pl.ds(st, CHS)])

    return k


RE = 2000

_S3 = 3.0 ** 0.5
_S5 = 5.0 ** 0.5
_S15 = 15.0 ** 0.5
_EMB_VALS = [MAX_RADIUS * (i + 1) / (NB + 1) for i in range(NB)]
_EMB_STEP = _EMB_VALS[1] - _EMB_VALS[0]
_EMB_SCALE = 1.14136 * math.exp(2.0) * (NB ** 0.5)


def _coef_body(nl, ev_ref, wr1_ref, br1_ref, *refs):
    wsh = refs[0:nl]
    wr2 = refs[nl:2 * nl]
    outs = refs[2 * nl:3 * nl]
    ev = ev_ref[:, 0:3]
    r2 = jnp.sum(ev * ev, axis=1, keepdims=True) + 1e-12
    r = jnp.sqrt(r2)
    u = ev / r
    x = u[:, 0:1]
    y = u[:, 1:2]
    z = u[:, 2:3]
    sh_list = [
        jnp.ones_like(x),
        _S3 * x, _S3 * y, _S3 * z,
        _S15 * x * y, _S15 * y * z, (_S5 / 2.0) * (3.0 * z * z - 1.0),
        _S15 * x * z, (_S15 / 2.0) * (x * x - y * y),
    ]
    sh16 = jnp.concatenate(sh_list + [jnp.zeros((RE, 7), jnp.float32)],
                           axis=1).astype(_BF)
    ii = lax.broadcasted_iota(jnp.int32, (RE, NB), 1).astype(jnp.float32)
    vals = ii * _EMB_STEP + _EMB_VALS[0]
    diff = (r - vals) / _EMB_STEP
    d2 = diff * diff
    inside = d2 < 1.0
    d2c = jnp.where(inside, d2, 0.0)
    emb = _EMB_SCALE * jnp.where(inside, jnp.exp(-1.0 / (1.0 - d2c)), 0.0)
    hidden = jnp.dot(emb.astype(_BF), wr1_ref[...],
                     preferred_element_type=jnp.float32) + br1_ref[...]
    hidden = hidden * jax.nn.sigmoid(hidden)
    for l in range(nl):
        hb = hidden[:, 128 * l:128 * (l + 1)].astype(_BF)
        w = jnp.dot(hb, wr2[l][...], preferred_element_type=jnp.float32)
        sha = jnp.dot(sh16, wsh[l][...], preferred_element_type=jnp.float32)
        prodv = sha * w
        outs[l][0, :, :] = prodv[:, :DW]
        outs[l][1, :, :] = prodv[:, DW:]


def _make_coef(nl):
    grid = (E // RE,)
    full = lambda shape: pl.BlockSpec(shape, lambda i: (0,) * len(shape))
    in_specs = [
        pl.BlockSpec((RE, 16), lambda i: (i, 0)),
        full((NB, 128 * nl)),
        full((1, 128 * nl)),
    ]
    in_specs += [full((16, 2 * DW)) for _ in range(nl)]
    in_specs += [full((128, 2 * DW)) for _ in range(nl)]
    out_specs = [pl.BlockSpec((2, RE, DW), lambda i: (0, i, 0))
                 for _ in range(nl)]
    out_shape = [jax.ShapeDtypeStruct((2, E, DW), jnp.float32)
                 for _ in range(nl)]
    return pl.pallas_call(
        functools.partial(_coef_body, nl), grid=grid, in_specs=in_specs,
        out_specs=out_specs, out_shape=out_shape)


RB = 1024


def _make_hf0(di):
    def body(x_ref, wfa_ref, wfb_ref, o_ref):
        xv = x_ref[...]
        o_ref[0, :, :] = xv @ wfa_ref[...]
        o_ref[1, :, :] = xv @ wfb_ref[...]

    return pl.pallas_call(
        body, grid=(NP // RB,),
        in_specs=[pl.BlockSpec((RB, di), lambda i: (i, 0)),
                  pl.BlockSpec((di, DW), lambda i: (0, 0)),
                  pl.BlockSpec((di, DW), lambda i: (0, 0))],
        out_specs=pl.BlockSpec((2, RB, DW), lambda i: (0, i, 0)),
        out_shape=jax.ShapeDtypeStruct((2, NP, DW), jnp.float32))


def _merge_agg(p_ref, do):
    dr = DREAL[do]
    return jnp.concatenate([p_ref[0, :, :dr], p_ref[1, :, :dr]], axis=1)


def _make_update(di, do):
    dh, dhn = DH[do], DW

    def body(h_ref, na_ref, p_ref, wsc_ref, wa_ref, wfa_ref, wfb_ref,
             h_out, hf_out):
        agg = _merge_agg(p_ref, do)
        scv = (h_ref[...] @ wsc_ref[...]) * (na_ref[...] @ wa_ref[...])
        hn = scv + agg
        hn = hn * jax.nn.sigmoid(hn)
        h_out[...] = hn
        hf_out[0, :, :] = hn @ wfa_ref[...]
        hf_out[1, :, :] = hn @ wfb_ref[...]

    return pl.pallas_call(
        body, grid=(NP // RB,),
        in_specs=[pl.BlockSpec((RB, di), lambda i: (i, 0)),
                  pl.BlockSpec((RB, D_ATTR), lambda i: (i, 0)),
                  pl.BlockSpec((2, RB, dh), lambda i: (0, i, 0)),
                  pl.BlockSpec((di, do), lambda i: (0, 0)),
                  pl.BlockSpec((D_ATTR, do), lambda i: (0, 0)),
                  pl.BlockSpec((do, dhn), lambda i: (0, 0)),
                  pl.BlockSpec((do, dhn), lambda i: (0, 0))],
        out_specs=[pl.BlockSpec((RB, do), lambda i: (i, 0)),
                   pl.BlockSpec((2, RB, dhn), lambda i: (0, i, 0))],
        out_shape=[jax.ShapeDtypeStruct((NP, do), jnp.float32),
                   jax.ShapeDtypeStruct((2, NP, dhn), jnp.float32)])


def _make_final(di, do):
    dh = DH[do]

    def body(h_ref, na_ref, p_ref, b_ref, wsc_ref, wa_ref, o_ref):
        i = pl.program_id(0)
        agg = _merge_agg(p_ref, do)
        scv = (h_ref[...] @ wsc_ref[...]) * (na_ref[...] @ wa_ref[...])
        hn = (scv + agg) * 0.01
        ids = b_ref[0, 0, :]
        onehot = (ids[:, None] ==
                  lax.broadcasted_iota(jnp.int32, (RB, NGRAPH), 1)
                  ).astype(jnp.float32)
        contrib = lax.dot_general(onehot, hn, (((0,), (0,)), ((), ())))

        @pl.when(i == 0)
        def _():
            o_ref[...] = jnp.zeros_like(o_ref)

        o_ref[...] += contrib

    return pl.pallas_call(
        body, grid=(NP // RB,),
        in_specs=[pl.BlockSpec((RB, di), lambda i: (i, 0)),
                  pl.BlockSpec((RB, D_ATTR), lambda i: (i, 0)),
                  pl.BlockSpec((2, RB, dh), lambda i: (0, i, 0)),
                  pl.BlockSpec((1, 1, RB), lambda i: (i, 0, 0)),
                  pl.BlockSpec((di, do), lambda i: (0, 0)),
                  pl.BlockSpec((D_ATTR, do), lambda i: (0, 0))],
        out_specs=pl.BlockSpec((NGRAPH, do), lambda i: (0, 0)),
        out_shape=jax.ShapeDtypeStruct((NGRAPH, do), jnp.float32))


def kernel(pos, x, node_attr, edge_index, batch, params):
    f32 = jnp.float32
    srcs = edge_index[0].astype(jnp.int32)
    dsts = edge_index[1].astype(jnp.int32)
    srcs2p = srcs.reshape(NCHUNK_P, CHP)
    dsts2p = dsts.reshape(NCHUNK_P, CHP)
    srcs2 = srcs.reshape(NCHUNK_S, CHS)
    dsts2 = dsts.reshape(NCHUNK_S, CHS)
    pos16 = jnp.zeros((NP, 16), f32).at[:N, :3].set(pos.astype(f32))
    x_p = jnp.zeros((NP, DIMS[0]), f32).at[:N].set(x.astype(f32))
    na_p = jnp.zeros((NP, D_ATTR), f32).at[:N].set(node_attr.astype(f32))
    batch_p = jnp.full((NP,), NGRAPH, jnp.int32).at[:N].set(
        batch.astype(jnp.int32))
    batch3d = batch_p.reshape(NP // RB, 1, RB)

    wsh_s, wr2_s, wf_s = [], [], []
    for l in range(NLAYERS):
        do = DIMS[l + 1]
        sa, sb = _split_pad_cols(params["Wsh%d" % l] * 0.25, do, DW)
        wsh_s.append(jnp.pad(jnp.concatenate([sa, sb], axis=1),
                             ((0, 7), (0, 0))).astype(_BF))
        ra, rb = _split_pad_cols(params["Wr2_%d" % l], do, DW)
        wr2_s.append(jnp.pad(jnp.concatenate([ra, rb], axis=1),
                             ((0, 28), (0, 0))).astype(_BF))
        wf_s.append(_split_pad_cols(params["Wf%d" % l], do, DW))
    wr1_ab = [jnp.concatenate(
        [jnp.pad(params["Wr1_%d" % l], ((0, 0), (0, 28)))
         for l in ls], axis=1).astype(_BF) for ls in ((0, 1), (2, 3))]
    br1_ab = [jnp.concatenate(
        [jnp.pad(params["br1_%d" % l], ((0, 28),)) for l in ls]
    ).reshape(1, 256) for ls in ((0, 1), (2, 3))]

    ev = _make_pos_gather()(pos16, srcs2p, dsts2p)
    coef01 = _make_coef(2)(ev, wr1_ab[0], br1_ab[0],
                           wsh_s[0], wsh_s[1], wr2_s[0], wr2_s[1])
    coef23 = _make_coef(2)(ev, wr1_ab[1], br1_ab[1],
                           wsh_s[2], wsh_s[3], wr2_s[2], wr2_s[3])
    coefs = list(coef01) + list(coef23)

    h = x_p
    hf2 = _make_hf0(DIMS[0])(x_p, *wf_s[0])
    for l in range(NLAYERS - 1):
        do = DIMS[l + 1]
        part = _make_sc_scatter(DH[do], DW)(hf2, coefs[l], srcs2, dsts2)
        h, hf2 = _make_update(DIMS[l], do)(
            h, na_p, part,
            params["Wsc%d" % l], params["Wa%d" % l], *wf_s[l + 1])
    do = DIMS[4]
    part = _make_sc_scatter(DH[do], DW)(hf2, coefs[3], srcs2, dsts2)
    out = _make_final(DIMS[3], do)(
        h, na_p, part, batch3d, params["Wsc3"], params["Wa3"])
    return out


# speedup vs baseline: 1.0139x; 1.0139x over previous
import functools
import math

import jax
import jax.numpy as jnp
from jax import lax
from jax.experimental import pallas as pl
from jax.experimental.pallas import tpu as pltpu
from jax.experimental.pallas import tpu_sc as plsc

N = 10000
E = 160000
NGRAPH = 16
NB = 10
MAX_RADIUS = 3.5
D_ATTR = 16

NP = 10240
CHP = 128
NCHUNK_P = E // CHP
NC = 2
NS = 16
ROWS_PER_TILE = NP // NS

DIMS = [128, 144, 144, 144, 64]
NLAYERS = 4
DH = {144: 80, 64: 32}
DREAL = {144: 72, 64: 32}
DW = 128

CHS = 80
NCHUNK_S = E // CHS
NKS = NCHUNK_S // NS


def _sc_mesh():
    return plsc.VectorSubcoreMesh(core_axis_name="c", subcore_axis_name="s")


_SC_PARAMS = pltpu.CompilerParams(use_tc_tiling_on_sc=False)
_SC_PARAMS_NL = pltpu.CompilerParams(use_tc_tiling_on_sc=False,
                                     needs_layout_passes=False)
_BF = jnp.bfloat16


def _split_pad_cols(w, do, dh):
    dr = DREAL[do]
    a = w[..., :dr]
    b = w[..., dr:do]
    pad = [(0, 0)] * (w.ndim - 1)
    a = jnp.pad(a, pad + [(0, dh - a.shape[-1])])
    b = jnp.pad(b, pad + [(0, dh - b.shape[-1])])
    return a, b


def _swizzle(w, dhc):
    perm = []
    for g in range(dhc // 32):
        for k in range(16):
            perm.append(32 * g + k)
            perm.append(32 * g + 16 + k)
    return w[..., jnp.array(perm, dtype=jnp.int32)]


NKP = NCHUNK_P // 32
REMP = NCHUNK_P - NKP * 32


def _make_pos_gather():
    @functools.partial(
        pl.kernel,
        out_type=jax.ShapeDtypeStruct((E, 16), jnp.float32),
        mesh=_sc_mesh(),
        scratch_types=[
            pltpu.VMEM((NKP + 1, CHP), jnp.int32),
            pltpu.VMEM((NKP + 1, CHP), jnp.int32),
            pltpu.VMEM((CHP, 16), jnp.float32),
            pltpu.VMEM((CHP, 16), jnp.float32),
            pltpu.VMEM((CHP, 16), jnp.float32),
            pltpu.VMEM((CHP, 16), jnp.float32),
            pltpu.VMEM((CHP, 16), jnp.float32),
            pltpu.VMEM((CHP, 16), jnp.float32),
            pltpu.SemaphoreType.DMA,
            pltpu.SemaphoreType.DMA,
            pltpu.SemaphoreType.DMA,
            pltpu.SemaphoreType.DMA,
            pltpu.SemaphoreType.DMA,
            pltpu.SemaphoreType.DMA,
        ],
        compiler_params=_SC_PARAMS,
    )
    def k(pos16, srcs2, dsts2, ev_out, src_all, dst_all,
          a0, a1, b0, b1, e0, e1, ga0, ga1, gb0, gb1, w0, w1):
        c = lax.axis_index("c")
        s = lax.axis_index("s")
        wid = s * NC + c
        abuf = (a0, a1)
        bbuf = (b0, b1)
        ebuf = (e0, e1)
        gsa = (ga0, ga1)
        gsb = (gb0, gb1)
        wsem = (w0, w1)
        cbase = wid * NKP + jnp.minimum(wid, REMP)
        nk = NKP + jnp.where(wid < REMP, 1, 0)
        pltpu.sync_copy(srcs2.at[pl.ds(cbase, NKP)],
                        src_all.at[pl.ds(0, NKP)])
        pltpu.sync_copy(dsts2.at[pl.ds(cbase, NKP)],
                        dst_all.at[pl.ds(0, NKP)])

        @pl.when(nk > NKP)
        def _():
            pltpu.sync_copy(srcs2.at[pl.ds(cbase + NKP, 1)],
                            src_all.at[pl.ds(NKP, 1)])
            pltpu.sync_copy(dsts2.at[pl.ds(cbase + NKP, 1)],
                            dst_all.at[pl.ds(NKP, 1)])

        def issue(kk, p):
            pltpu.async_copy(pos16.at[src_all.at[kk]], abuf[p], gsa[p])
            pltpu.async_copy(pos16.at[dst_all.at[kk]], bbuf[p], gsb[p])

        issue(0, 0)
        issue(1, 1)
        dummy = pos16.at[pl.ds(0, CHP)]

        def pair(q, _):
            for p in range(2):
                kk = q * 2 + p

                @pl.when(kk < nk)
                def _(kk=kk, p=p):
                    pltpu.make_async_copy(dummy, abuf[p], gsa[p]).wait()
                    pltpu.make_async_copy(dummy, bbuf[p], gsb[p]).wait()

                    @pl.when(kk >= 2)
                    def _():
                        pltpu.make_async_copy(dummy, ebuf[p], wsem[p]).wait()

                    @plsc.parallel_loop(0, CHP, 1, unroll=4)
                    def _(i):
                        sl = pl.ds(0, 16)
                        ebuf[p][i, sl] = abuf[p][i, sl] - bbuf[p][i, sl]

                    pltpu.async_copy(
                        ebuf[p], ev_out.at[pl.ds((cbase + kk) * CHP, CHP)],
                        wsem[p])

                    @pl.when(kk + 2 < nk)
                    def _():
                        issue(kk + 2, p)
            return 0

        lax.fori_loop(0, (nk + 1) // 2, pair, 0)
        for p in range(2):
            pltpu.make_async_copy(dummy, ebuf[p], wsem[p]).wait()

    return k


def _make_sc_scatter(dh, dwc):
    @functools.partial(
        pl.kernel,
        out_type=jax.ShapeDtypeStruct((NC, NP, dh), jnp.float32),
        mesh=_sc_mesh(),
        scratch_types=[
            pltpu.VMEM((NKS, CHS), jnp.int32),
            pltpu.VMEM((NKS, CHS), jnp.int32),
            pltpu.VMEM((CHS, DW), jnp.float32),
            pltpu.VMEM((CHS, DW), jnp.float32),
            pltpu.VMEM((CHS, dwc), jnp.float32),
            pltpu.VMEM((CHS, dwc), jnp.float32),
            pltpu.VMEM((CHS, dh), jnp.float32),
            pltpu.VMEM((CHS, dh), jnp.float32),
            pltpu.VMEM((16, dh), jnp.float32),
            pltpu.VMEM_SHARED((NP, dh), jnp.float32),
            pltpu.SemaphoreType.DMA,
            pltpu.SemaphoreType.DMA,
            pltpu.SemaphoreType.DMA,
            pltpu.SemaphoreType.DMA,
            pltpu.SemaphoreType.DMA,
            pltpu.SemaphoreType.DMA,
        ],
        compiler_params=_SC_PARAMS,
    )
    def k(hf2, coef2, srcs2, dsts2, out, src_all, dst_all,
          r0, r1, cl0, cl1, p0, p1, zbuf, aggsh,
          g0, g1, q0, q1, s0, s1):
        c = lax.axis_index("c")
        s = lax.axis_index("s")
        rows = (r0, r1)
        cld = (cl0, cl1)
        prod = (p0, p1)
        gsem = (g0, g1)
        csem = (q0, q1)
        ssem = (s0, s1)
        cbase = s * NKS

        pltpu.sync_copy(srcs2.at[pl.ds(cbase, NKS)], src_all)
        pltpu.sync_copy(dsts2.at[pl.ds(cbase, NKS)], dst_all)

        zero16 = jnp.zeros((16,), jnp.float32)

        def zb(i, _):
            for j in range(dh // 16):
                zbuf[i, pl.ds(j * 16, 16)] = zero16
            return 0

        lax.fori_loop(0, 16, zb, 0)
        for j in range(ROWS_PER_TILE // 16):
            pltpu.sync_copy(zbuf, aggsh.at[pl.ds(s * ROWS_PER_TILE + j * 16, 16)])
        plsc.subcore_barrier()

        def issue(kk, b):
            pltpu.async_copy(hf2.at[c].at[src_all.at[kk]], rows[b], gsem[b])
            pltpu.async_copy(coef2.at[c, pl.ds((cbase + kk) * CHS, CHS)],
                             cld[b], csem[b])

        issue(0, 0)
        issue(1, 1)
        dummy = hf2.at[c, pl.ds(0, CHS)]
        dummyc = coef2.at[c, pl.ds(0, CHS)]
        dummyn = out.at[c, pl.ds(0, CHS)]

        def pair(q, _):
            for b in range(2):
                kk = q * 2 + b

                @pl.when(kk < NKS)
                def _(kk=kk, b=b):
                    pltpu.make_async_copy(dummy, rows[b], gsem[b]).wait()
                    pltpu.make_async_copy(dummyc, cld[b], csem[b]).wait()

                    @pl.when(kk >= 2)
                    def _():
                        pltpu.make_async_copy(dummyn, prod[b], ssem[b]).wait()

                    @plsc.parallel_loop(0, CHS, 1, unroll=4)
                    def _(i):
                        for j in range(dh // 16):
                            sl = pl.ds(j * 16, 16)
                            prod[b][i, sl] = rows[b][i, sl] * cld[b][i, sl]

                    pltpu.async_copy(prod[b], aggsh.at[dst_all.at[kk]],
                                     ssem[b], add=True)

                    @pl.when(kk + 2 < NKS)
                    def _():
                        issue(kk + 2, b)
            return 0

        lax.fori_loop(0, (NKS + 1) // 2, pair, 0)
        for b in range(2):
            pltpu.make_async_copy(dummyn, prod[b], ssem[b]).wait()
        plsc.subcore_barrier()
        for j in range(ROWS_PER_TILE // CHS):
            st = s * ROWS_PER_TILE + j * CHS
            pltpu.sync_copy(aggsh.at[pl.ds(st, CHS)], p0)
            pltpu.sync_copy(p0, out.at[c, pl.ds(st, CHS)])

    return k


RE = 2000

_S3 = 3.0 ** 0.5
_S5 = 5.0 ** 0.5
_S15 = 15.0 ** 0.5
_EMB_VALS = [MAX_RADIUS * (i + 1) / (NB + 1) for i in range(NB)]
_EMB_STEP = _EMB_VALS[1] - _EMB_VALS[0]
_EMB_SCALE = 1.14136 * math.exp(2.0) * (NB ** 0.5)


def _coef_body(nl, ev_ref, wr1_ref, br1_ref, *refs):
    wsh = refs[0:nl]
    wr2 = refs[nl:2 * nl]
    outs = refs[2 * nl:3 * nl]
    ev = ev_ref[:, 0:3]
    r2 = jnp.sum(ev * ev, axis=1, keepdims=True) + 1e-12
    r = jnp.sqrt(r2)
    u = ev / r
    x = u[:, 0:1]
    y = u[:, 1:2]
    z = u[:, 2:3]
    sh_list = [
        jnp.ones_like(x),
        _S3 * x, _S3 * y, _S3 * z,
        _S15 * x * y, _S15 * y * z, (_S5 / 2.0) * (3.0 * z * z - 1.0),
        _S15 * x * z, (_S15 / 2.0) * (x * x - y * y),
    ]
    sh16 = jnp.concatenate(sh_list + [jnp.zeros((RE, 7), jnp.float32)],
                           axis=1).astype(_BF)
    ii = lax.broadcasted_iota(jnp.int32, (RE, NB), 1).astype(jnp.float32)
    vals = ii * _EMB_STEP + _EMB_VALS[0]
    diff = (r - vals) / _EMB_STEP
    d2 = diff * diff
    inside = d2 < 1.0
    d2c = jnp.where(inside, d2, 0.0)
    emb = _EMB_SCALE * jnp.where(inside, jnp.exp(-1.0 / (1.0 - d2c)), 0.0)
    hidden = jnp.dot(emb.astype(_BF), wr1_ref[...],
                     preferred_element_type=jnp.float32) + br1_ref[...]
    hidden = hidden * jax.nn.sigmoid(hidden)
    for l in range(nl):
        hb = hidden[:, 128 * l:128 * (l + 1)].astype(_BF)
        w = jnp.dot(hb, wr2[l][...], preferred_element_type=jnp.float32)
        sha = jnp.dot(sh16, wsh[l][...], preferred_element_type=jnp.float32)
        prodv = sha * w
        outs[l][0, :, :] = prodv[:, :DW]
        outs[l][1, :, :] = prodv[:, DW:]


def _make_coef(nl):
    grid = (E // RE,)
    full = lambda shape: pl.BlockSpec(shape, lambda i: (0,) * len(shape))
    in_specs = [
        pl.BlockSpec((RE, 16), lambda i: (i, 0)),
        full((NB, 128 * nl)),
        full((1, 128 * nl)),
    ]
    in_specs += [full((16, 2 * DW)) for _ in range(nl)]
    in_specs += [full((128, 2 * DW)) for _ in range(nl)]
    out_specs = [pl.BlockSpec((2, RE, DW), lambda i: (0, i, 0))
                 for _ in range(nl)]
    out_shape = [jax.ShapeDtypeStruct((2, E, DW), jnp.float32)
                 for _ in range(nl)]
    return pl.pallas_call(
        functools.partial(_coef_body, nl), grid=grid, in_specs=in_specs,
        out_specs=out_specs, out_shape=out_shape)


RB = 1024


def _make_hf0(di):
    def body(x_ref, wfa_ref, wfb_ref, o_ref):
        xv = x_ref[...]
        o_ref[0, :, :] = xv @ wfa_ref[...]
        o_ref[1, :, :] = xv @ wfb_ref[...]

    return pl.pallas_call(
        body, grid=(NP // RB,),
        in_specs=[pl.BlockSpec((RB, di), lambda i: (i, 0)),
                  pl.BlockSpec((di, DW), lambda i: (0, 0)),
                  pl.BlockSpec((di, DW), lambda i: (0, 0))],
        out_specs=pl.BlockSpec((2, RB, DW), lambda i: (0, i, 0)),
        out_shape=jax.ShapeDtypeStruct((2, NP, DW), jnp.float32))


def _merge_agg(p_ref, do):
    dr = DREAL[do]
    return jnp.concatenate([p_ref[0, :, :dr], p_ref[1, :, :dr]], axis=1)


def _make_update(di, do):
    dh, dhn = DH[do], DW

    def body(h_ref, na_ref, p_ref, wsc_ref, wa_ref, wfa_ref, wfb_ref,
             h_out, hf_out):
        agg = _merge_agg(p_ref, do)
        scv = (h_ref[...] @ wsc_ref[...]) * (na_ref[...] @ wa_ref[...])
        hn = scv + agg
        hn = hn * jax.nn.sigmoid(hn)
        h_out[...] = hn
        hf_out[0, :, :] = hn @ wfa_ref[...]
        hf_out[1, :, :] = hn @ wfb_ref[...]

    return pl.pallas_call(
        body, grid=(NP // RB,),
        in_specs=[pl.BlockSpec((RB, di), lambda i: (i, 0)),
                  pl.BlockSpec((RB, D_ATTR), lambda i: (i, 0)),
                  pl.BlockSpec((2, RB, dh), lambda i: (0, i, 0)),
                  pl.BlockSpec((di, do), lambda i: (0, 0)),
                  pl.BlockSpec((D_ATTR, do), lambda i: (0, 0)),
                  pl.BlockSpec((do, dhn), lambda i: (0, 0)),
                  pl.BlockSpec((do, dhn), lambda i: (0, 0))],
        out_specs=[pl.BlockSpec((RB, do), lambda i: (i, 0)),
                   pl.BlockSpec((2, RB, dhn), lambda i: (0, i, 0))],
        out_shape=[jax.ShapeDtypeStruct((NP, do), jnp.float32),
                   jax.ShapeDtypeStruct((2, NP, dhn), jnp.float32)])


def _make_final(di, do):
    dh = DH[do]

    def body(h_ref, na_ref, p_ref, b_ref, wsc_ref, wa_ref, o_ref):
        i = pl.program_id(0)
        agg = _merge_agg(p_ref, do)
        scv = (h_ref[...] @ wsc_ref[...]) * (na_ref[...] @ wa_ref[...])
        hn = (scv + agg) * 0.01
        ids = b_ref[0, 0, :]
        onehot = (ids[:, None] ==
                  lax.broadcasted_iota(jnp.int32, (RB, NGRAPH), 1)
                  ).astype(jnp.float32)
        contrib = lax.dot_general(onehot, hn, (((0,), (0,)), ((), ())))

        @pl.when(i == 0)
        def _():
            o_ref[...] = jnp.zeros_like(o_ref)

        o_ref[...] += contrib

    return pl.pallas_call(
        body, grid=(NP // RB,),
        in_specs=[pl.BlockSpec((RB, di), lambda i: (i, 0)),
                  pl.BlockSpec((RB, D_ATTR), lambda i: (i, 0)),
                  pl.BlockSpec((2, RB, dh), lambda i: (0, i, 0)),
                  pl.BlockSpec((1, 1, RB), lambda i: (i, 0, 0)),
                  pl.BlockSpec((di, do), lambda i: (0, 0)),
                  pl.BlockSpec((D_ATTR, do), lambda i: (0, 0))],
        out_specs=pl.BlockSpec((NGRAPH, do), lambda i: (0, 0)),
        out_shape=jax.ShapeDtypeStruct((NGRAPH, do), jnp.float32))


def kernel(pos, x, node_attr, edge_index, batch, params):
    f32 = jnp.float32
    srcs = edge_index[0].astype(jnp.int32)
    dsts = edge_index[1].astype(jnp.int32)
    srcs2p = srcs.reshape(NCHUNK_P, CHP)
    dsts2p = dsts.reshape(NCHUNK_P, CHP)
    srcs2 = srcs.reshape(NCHUNK_S, CHS)
    dsts2 = dsts.reshape(NCHUNK_S, CHS)
    pos16 = jnp.zeros((NP, 16), f32).at[:N, :3].set(pos.astype(f32))
    x_p = jnp.zeros((NP, DIMS[0]), f32).at[:N].set(x.astype(f32))
    na_p = jnp.zeros((NP, D_ATTR), f32).at[:N].set(node_attr.astype(f32))
    batch_p = jnp.full((NP,), NGRAPH, jnp.int32).at[:N].set(
        batch.astype(jnp.int32))
    batch3d = batch_p.reshape(NP // RB, 1, RB)

    wsh_s, wr2_s, wf_s = [], [], []
    for l in range(NLAYERS):
        do = DIMS[l + 1]
        sa, sb = _split_pad_cols(params["Wsh%d" % l] * 0.25, do, DW)
        wsh_s.append(jnp.pad(jnp.concatenate([sa, sb], axis=1),
                             ((0, 7), (0, 0))).astype(_BF))
        ra, rb = _split_pad_cols(params["Wr2_%d" % l], do, DW)
        wr2_s.append(jnp.pad(jnp.concatenate([ra, rb], axis=1),
                             ((0, 28), (0, 0))).astype(_BF))
        wf_s.append(_split_pad_cols(params["Wf%d" % l], do, DW))
    wr1_all = jnp.concatenate(
        [jnp.pad(params["Wr1_%d" % l], ((0, 0), (0, 28)))
         for l in range(NLAYERS)], axis=1).astype(_BF)
    br1_all = jnp.concatenate(
        [jnp.pad(params["br1_%d" % l], ((0, 28),)) for l in range(NLAYERS)]
    ).reshape(1, 512)

    ev = _make_pos_gather()(pos16, srcs2p, dsts2p)
    coefs = _make_coef(4)(ev, wr1_all, br1_all, *wsh_s, *wr2_s)

    h = x_p
    hf2 = _make_hf0(DIMS[0])(x_p, *wf_s[0])
    for l in range(NLAYERS - 1):
        do = DIMS[l + 1]
        part = _make_sc_scatter(DH[do], DW)(hf2, coefs[l], srcs2, dsts2)
        h, hf2 = _make_update(DIMS[l], do)(
            h, na_p, part,
            params["Wsc%d" % l], params["Wa%d" % l], *wf_s[l + 1])
    do = DIMS[4]
    part = _make_sc_scatter(DH[do], DW)(hf2, coefs[3], srcs2, dsts2)
    out = _make_final(DIMS[3], do)(
        h, na_p, part, batch3d, params["Wsc3"], params["Wa3"])
    return out
